# baseline (speedup 1.0000x reference)
import functools

import jax
import jax.numpy as jnp
from jax import lax
from jax.experimental import pallas as pl
from jax.experimental.pallas import tpu as pltpu
from jax.experimental.pallas import tpu_sc as plsc

NC = 2
NS = 16
NW = NC * NS
LANES = 16
EPS = 1e-12


def _rot(v, sh):
    idx = (lax.iota(jnp.int32, LANES) + jnp.int32(sh)) & jnp.int32(LANES - 1)
    dnums = lax.GatherDimensionNumbers(
        offset_dims=(), collapsed_slice_dims=(0,), start_index_map=(0,))
    return lax.gather(v, idx[:, None], dnums, slice_sizes=(1,),
                      mode=lax.GatherScatterMode.PROMISE_IN_BOUNDS)


def _make_sc_kernel(n_tok, seq, hidden, chunk):
    n_per_w = n_tok // NW
    n_chunks = n_per_w // chunk
    n_pairs = n_chunks // 2
    n_f = hidden // LANES
    mesh = plsc.VectorSubcoreMesh(core_axis_name="c", subcore_axis_name="s")

    @functools.partial(
        pl.kernel,
        mesh=mesh,
        out_type=jax.ShapeDtypeStruct((n_tok, hidden), jnp.float32),
        scratch_types=[
            pltpu.VMEM((n_per_w,), jnp.int32),
            pltpu.VMEM((chunk, hidden), jnp.float32),
            pltpu.VMEM((chunk, hidden), jnp.float32),
            pltpu.VMEM((chunk, hidden), jnp.float32),
            pltpu.VMEM((chunk, hidden), jnp.float32),
            pltpu.VMEM((hidden,), jnp.float32),
            pltpu.VMEM((hidden,), jnp.float32),
            pltpu.VMEM((chunk * LANES,), jnp.float32),
            pltpu.VMEM((chunk * LANES,), jnp.float32),
            pltpu.VMEM((chunk * LANES,), jnp.float32),
            pltpu.VMEM((chunk * LANES,), jnp.float32),
            pltpu.SemaphoreType.DMA,
            pltpu.SemaphoreType.DMA,
            pltpu.SemaphoreType.DMA,
            pltpu.SemaphoreType.DMA,
        ],
    )
    def sc_kernel(ids_hbm, wtab_hbm, ptab_hbm, g_hbm, b_hbm, out_hbm,
                  idxs, wrow_a, wrow_b, prow_a, prow_b, gbuf, bbuf,
                  sbuf, qbuf, avbuf, m2buf,
                  sem_ga, sem_gb, sem_pa, sem_pb):
        wid = lax.axis_index("s") * NC + lax.axis_index("c")
        base_tok = wid * n_per_w

        pltpu.sync_copy(ids_hbm.at[pl.ds(base_tok, n_per_w)], idxs)
        pltpu.sync_copy(g_hbm, gbuf)
        pltpu.sync_copy(b_hbm, bbuf)

        def start_fetch(k, wrow, prow, sem_g, sem_p):
            tok0 = base_tok + k * chunk
            pltpu.async_copy(
                wtab_hbm.at[idxs.at[pl.ds(k * chunk, chunk)]], wrow, sem_g)
            s0 = lax.rem(tok0, seq)
            pltpu.async_copy(ptab_hbm.at[pl.ds(s0, chunk)], prow, sem_p)

        def wait_fetch(k, wrow, prow, sem_g, sem_p):
            pltpu.make_async_copy(
                wtab_hbm.at[idxs.at[pl.ds(k * chunk, chunk)]], wrow,
                sem_g).wait()
            s0 = lax.rem(base_tok + k * chunk, seq)
            pltpu.make_async_copy(
                ptab_hbm.at[pl.ds(s0, chunk)], prow, sem_p).wait()

        def compute_chunk(k, wrow, prow):
            def token_body(t):
                zero = lax.iota(jnp.int32, LANES) * jnp.int32(0)
                z = lax.convert_element_type(zero, jnp.float32)
                accs = [z, z, z, z]
                accq = [z, z, z, z]
                for f in range(n_f):
                    w = wrow[t, pl.ds(f * LANES, LANES)]
                    p = prow[t, pl.ds(f * LANES, LANES)]
                    e = w + p
                    wrow[t, pl.ds(f * LANES, LANES)] = e
                    accs[f % 4] = accs[f % 4] + e
                    accq[f % 4] = accq[f % 4] + e * e
                sbuf[pl.ds(t * LANES, LANES)] = \
                    (accs[0] + accs[1]) + (accs[2] + accs[3])
                qbuf[pl.ds(t * LANES, LANES)] = \
                    (accq[0] + accq[1]) + (accq[2] + accq[3])

            plsc.parallel_loop(0, chunk, unroll=2)(token_body)

            def stats_body(t):
                sv = sbuf[pl.ds(t * LANES, LANES)]
                qv = qbuf[pl.ds(t * LANES, LANES)]
                for sh in (8, 4, 2, 1):
                    sv = sv + _rot(sv, sh)
                    qv = qv + _rot(qv, sh)
                mv = sv * (1.0 / hidden)
                x = qv * (1.0 / hidden) - mv * mv + EPS
                xb = lax.bitcast_convert_type(x, jnp.int32)
                yi = jnp.int32(0x5F3759DF) - (xb >> jnp.int32(1))
                y = lax.bitcast_convert_type(yi, jnp.float32)
                y = y * (1.5 - 0.5 * x * y * y)
                y = y * (1.5 - 0.5 * x * y * y)
                av = y * (1.5 - 0.5 * x * y * y)
                avbuf[pl.ds(t * LANES, LANES)] = av
                m2buf[pl.ds(t * LANES, LANES)] = mv * av

            plsc.parallel_loop(0, chunk, unroll=4)(stats_body)

            def norm_t(t):
                av = avbuf[pl.ds(t * LANES, LANES)]
                m2 = m2buf[pl.ds(t * LANES, LANES)]
                for f in range(n_f):
                    e = wrow[t, pl.ds(f * LANES, LANES)]
                    g = gbuf[pl.ds(f * LANES, LANES)]
                    b = bbuf[pl.ds(f * LANES, LANES)]
                    wrow[t, pl.ds(f * LANES, LANES)] = (e * av - m2) * g + b

            plsc.parallel_loop(0, chunk, unroll=3)(norm_t)

            tok0 = base_tok + k * chunk
            pltpu.sync_copy(wrow, out_hbm.at[pl.ds(tok0, chunk)])

        start_fetch(0, wrow_a, prow_a, sem_ga, sem_pa)

        def pair_body(k2, _):
            ka = 2 * k2
            kb = 2 * k2 + 1
            start_fetch(kb, wrow_b, prow_b, sem_gb, sem_pb)
            wait_fetch(ka, wrow_a, prow_a, sem_ga, sem_pa)
            compute_chunk(ka, wrow_a, prow_a)

            @pl.when(k2 < n_pairs - 1)
            def _():
                start_fetch(ka + 2, wrow_a, prow_a, sem_ga, sem_pa)

            wait_fetch(kb, wrow_b, prow_b, sem_gb, sem_pb)
            compute_chunk(kb, wrow_b, prow_b)
            return ()

        lax.fori_loop(0, n_pairs, pair_body, (), unroll=False)

    return sc_kernel


def kernel(input_ids, word_embeddings, position_embeddings, ln_gamma, ln_beta):
    batch, seq = input_ids.shape
    hidden = word_embeddings.shape[1]
    n_tok = batch * seq
    ids_flat = input_ids.reshape(-1).astype(jnp.int32)
    sc = _make_sc_kernel(n_tok, seq, hidden, chunk=32)
    out = sc(ids_flat, word_embeddings, position_embeddings, ln_gamma, ln_beta)
    return out.reshape(batch, seq, hidden)


# speedup vs baseline: 1.0790x; 1.0790x over previous
import functools

import jax
import jax.numpy as jnp
from jax import lax
from jax.experimental import pallas as pl
from jax.experimental.pallas import tpu as pltpu
from jax.experimental.pallas import tpu_sc as plsc

NC = 2
NS = 16
NW = NC * NS
LANES = 16
EPS = 1e-12


def _rot(v, sh):
    idx = (lax.iota(jnp.int32, LANES) + jnp.int32(sh)) & jnp.int32(LANES - 1)
    dnums = lax.GatherDimensionNumbers(
        offset_dims=(), collapsed_slice_dims=(0,), start_index_map=(0,))
    return lax.gather(v, idx[:, None], dnums, slice_sizes=(1,),
                      mode=lax.GatherScatterMode.PROMISE_IN_BOUNDS)


def _make_sc_kernel(n_tok, seq, hidden, chunk):
    n_per_w = n_tok // NW
    n_chunks = n_per_w // chunk
    n_pairs = n_chunks // 2
    n_f = hidden // LANES
    mesh = plsc.VectorSubcoreMesh(core_axis_name="c", subcore_axis_name="s")

    @functools.partial(
        pl.kernel,
        mesh=mesh,
        out_type=jax.ShapeDtypeStruct((n_tok, hidden), jnp.float32),
        scratch_types=[
            pltpu.VMEM((n_per_w,), jnp.int32),
            pltpu.VMEM((chunk, hidden), jnp.float32),
            pltpu.VMEM((chunk, hidden), jnp.float32),
            pltpu.VMEM((chunk, hidden), jnp.float32),
            pltpu.VMEM((chunk, hidden), jnp.float32),
            pltpu.VMEM((hidden,), jnp.float32),
            pltpu.VMEM((hidden,), jnp.float32),
            pltpu.VMEM((chunk * LANES,), jnp.float32),
            pltpu.VMEM((chunk * LANES,), jnp.float32),
            pltpu.VMEM((chunk * LANES,), jnp.float32),
            pltpu.VMEM((chunk * LANES,), jnp.float32),
            pltpu.SemaphoreType.DMA,
            pltpu.SemaphoreType.DMA,
            pltpu.SemaphoreType.DMA,
            pltpu.SemaphoreType.DMA,
        ],
    )
    def sc_kernel(ids_hbm, wtab_hbm, ptab_hbm, g_hbm, b_hbm, out_hbm,
                  idxs, wrow_a, wrow_b, prow_a, prow_b, gbuf, bbuf,
                  sbuf, qbuf, avbuf, m2buf,
                  sem_ga, sem_gb, sem_pa, sem_pb):
        wid = lax.axis_index("s") * NC + lax.axis_index("c")
        base_tok = wid * n_per_w

        pltpu.sync_copy(ids_hbm.at[pl.ds(base_tok, n_per_w)], idxs)
        pltpu.sync_copy(g_hbm, gbuf)
        pltpu.sync_copy(b_hbm, bbuf)

        def start_fetch(k, wrow, prow, sem_g, sem_p):
            tok0 = base_tok + k * chunk
            pltpu.async_copy(
                wtab_hbm.at[idxs.at[pl.ds(k * chunk, chunk)]], wrow, sem_g)
            s0 = lax.rem(tok0, seq)
            pltpu.async_copy(ptab_hbm.at[pl.ds(s0, chunk)], prow, sem_p)

        def wait_fetch(k, wrow, prow, sem_g, sem_p):
            pltpu.make_async_copy(
                wtab_hbm.at[idxs.at[pl.ds(k * chunk, chunk)]], wrow,
                sem_g).wait()
            s0 = lax.rem(base_tok + k * chunk, seq)
            pltpu.make_async_copy(
                ptab_hbm.at[pl.ds(s0, chunk)], prow, sem_p).wait()

        def compute_chunk(k, wrow, prow):
            def token_body(t):
                zero = lax.iota(jnp.int32, LANES) * jnp.int32(0)
                z = lax.convert_element_type(zero, jnp.float32)
                accs = [z, z, z, z]
                accq = [z, z, z, z]
                for f in range(n_f):
                    w = wrow[t, pl.ds(f * LANES, LANES)]
                    p = prow[t, pl.ds(f * LANES, LANES)]
                    e = w + p
                    wrow[t, pl.ds(f * LANES, LANES)] = e
                    accs[f % 4] = accs[f % 4] + e
                    accq[f % 4] = accq[f % 4] + e * e
                sbuf[pl.ds(t * LANES, LANES)] = \
                    (accs[0] + accs[1]) + (accs[2] + accs[3])
                qbuf[pl.ds(t * LANES, LANES)] = \
                    (accq[0] + accq[1]) + (accq[2] + accq[3])

            plsc.parallel_loop(0, chunk, unroll=2)(token_body)

            def stats_body(t):
                sv = sbuf[pl.ds(t * LANES, LANES)]
                qv = qbuf[pl.ds(t * LANES, LANES)]
                for sh in (8, 4, 2, 1):
                    sv = sv + _rot(sv, sh)
                    qv = qv + _rot(qv, sh)
                mv = sv * (1.0 / hidden)
                x = qv * (1.0 / hidden) - mv * mv + EPS
                xb = lax.bitcast_convert_type(x, jnp.int32)
                yi = jnp.int32(0x5F3759DF) - (xb >> jnp.int32(1))
                y = lax.bitcast_convert_type(yi, jnp.float32)
                y = y * (1.5 - 0.5 * x * y * y)
                y = y * (1.5 - 0.5 * x * y * y)
                av = y * (1.5 - 0.5 * x * y * y)
                avbuf[pl.ds(t * LANES, LANES)] = av
                m2buf[pl.ds(t * LANES, LANES)] = mv * av

            plsc.parallel_loop(0, chunk, unroll=4)(stats_body)

            def norm_t(t):
                av = avbuf[pl.ds(t * LANES, LANES)]
                m2 = m2buf[pl.ds(t * LANES, LANES)]
                for f in range(n_f):
                    e = wrow[t, pl.ds(f * LANES, LANES)]
                    g = gbuf[pl.ds(f * LANES, LANES)]
                    b = bbuf[pl.ds(f * LANES, LANES)]
                    wrow[t, pl.ds(f * LANES, LANES)] = (e * av - m2) * g + b

            plsc.parallel_loop(0, chunk, unroll=2)(norm_t)

            tok0 = base_tok + k * chunk
            pltpu.sync_copy(wrow, out_hbm.at[pl.ds(tok0, chunk)])

        start_fetch(0, wrow_a, prow_a, sem_ga, sem_pa)

        def pair_body(k2, _):
            ka = 2 * k2
            kb = 2 * k2 + 1
            start_fetch(kb, wrow_b, prow_b, sem_gb, sem_pb)
            wait_fetch(ka, wrow_a, prow_a, sem_ga, sem_pa)
            compute_chunk(ka, wrow_a, prow_a)

            @pl.when(k2 < n_pairs - 1)
            def _():
                start_fetch(ka + 2, wrow_a, prow_a, sem_ga, sem_pa)

            wait_fetch(kb, wrow_b, prow_b, sem_gb, sem_pb)
            compute_chunk(kb, wrow_b, prow_b)
            return ()

        lax.fori_loop(0, n_pairs, pair_body, (), unroll=False)

    return sc_kernel


def kernel(input_ids, word_embeddings, position_embeddings, ln_gamma, ln_beta):
    batch, seq = input_ids.shape
    hidden = word_embeddings.shape[1]
    n_tok = batch * seq
    ids_flat = input_ids.reshape(-1).astype(jnp.int32)
    sc = _make_sc_kernel(n_tok, seq, hidden, chunk=32)
    out = sc(ids_flat, word_embeddings, position_embeddings, ln_gamma, ln_beta)
    return out.reshape(batch, seq, hidden)
